# baseline (speedup 1.0000x reference)
import jax
import jax.numpy as jnp
from jax import lax
from jax.experimental import pallas as pl
from jax.experimental.pallas import tpu as pltpu
from jax.experimental.pallas import tpu_sc as plsc

_VOCAB = 100000
_EMBED_DIM = 64
_BATCH = 4096
_HIST = 50

_NC = 2
_NS = 16
_NW = _NC * _NS

_BT = _BATCH // _NW
_MPB = _HIST * _EMBED_DIM
_ROWS = _BATCH * _MPB // 128


def _gather_body(table_hbm, idx_hbm, out_hbm, idx_v, buf0, buf1,
                 sem_g0, sem_g1, sem_w0, sem_w1):
  wid = lax.axis_index("s") * _NC + lax.axis_index("c")
  pltpu.sync_copy(idx_hbm.at[pl.ds(0, _HIST), pl.ds(wid * _BT, _BT)], idx_v)

  base = wid * _HIST // 2 * _BT

  def dst(h):
    return out_hbm.at[pl.ds(base + (h // 2) * _BT, _BT),
                      pl.ds((h % 2) * _EMBED_DIM, _EMBED_DIM)]

  pltpu.async_copy(table_hbm.at[idx_v.at[0]], buf0, sem_g0)

  def step(h, buf, sem_g, sem_w, first):
    pltpu.make_async_copy(table_hbm.at[idx_v.at[h]], buf, sem_g).wait()

    @pl.when(jnp.logical_not(first))
    def _drain_prev_write():
      pltpu.make_async_copy(buf, dst(h - 2), sem_w).wait()

    pltpu.async_copy(buf, dst(h), sem_w)

  def superstep(i, carry):
    h0 = 2 * i
    h1 = 2 * i + 1
    pltpu.async_copy(table_hbm.at[idx_v.at[h1]], buf1, sem_g1)
    step(h0, buf0, sem_g0, sem_w0, i == 0)

    @pl.when(i < _HIST // 2 - 1)
    def _fire_next():
      pltpu.async_copy(table_hbm.at[idx_v.at[h1 + 1]], buf0, sem_g0)

    step(h1, buf1, sem_g1, sem_w1, i == 0)
    return carry

  lax.fori_loop(0, _HIST // 2, superstep, 0)
  pltpu.make_async_copy(buf0, dst(_HIST - 2), sem_w0).wait()
  pltpu.make_async_copy(buf1, dst(_HIST - 1), sem_w1).wait()


def _tc_transpose_body(x_ref, y_ref):
  x = x_ref[...].reshape(_MPB // 128, 128, 128)
  y_ref[...] = jnp.swapaxes(x, 1, 2).reshape(_MPB, 128)


@jax.jit
def kernel(indices, table):
  idx_t = indices.T.astype(jnp.int32)
  mesh = plsc.VectorSubcoreMesh(core_axis_name="c", subcore_axis_name="s")
  staged = pl.kernel(
      _gather_body,
      out_type=jax.ShapeDtypeStruct((_ROWS, 128), jnp.float32),
      mesh=mesh,
      scratch_types=[
          pltpu.VMEM((_HIST, _BT), jnp.int32),
          pltpu.VMEM((_BT, _EMBED_DIM), jnp.float32),
          pltpu.VMEM((_BT, _EMBED_DIM), jnp.float32),
          pltpu.SemaphoreType.DMA,
          pltpu.SemaphoreType.DMA,
          pltpu.SemaphoreType.DMA,
          pltpu.SemaphoreType.DMA,
      ],
      compiler_params=pltpu.CompilerParams(use_tc_tiling_on_sc=False,
                                           needs_layout_passes=False),
  )(table, idx_t)

  trans = pl.pallas_call(
      _tc_transpose_body,
      grid=(_NW,),
      in_specs=[pl.BlockSpec((_MPB, 128), lambda w: (w, 0))],
      out_specs=pl.BlockSpec((_MPB, 128), lambda w: (0, w)),
      out_shape=jax.ShapeDtypeStruct((_MPB, _BATCH), jnp.float32),
  )(staged)

  return trans.reshape(_HIST, _EMBED_DIM, _BATCH).transpose(2, 0, 1)
